# baseline (speedup 1.0000x reference)
import functools

import jax
import jax.numpy as jnp
from jax import lax
from jax.experimental import pallas as pl
from jax.experimental.pallas import tpu as pltpu
from jax.experimental.pallas import tpu_sc as plsc

B, S, D = 4, 8192, 1024
NC, NS, L = 2, 16, 16
NW = NC * NS
ROWS_W = S // NW
CH = 16
RPC = CH * (D // 128)
N_CHUNKS = ROWS_W // CH
XR = B * S * (D // 128)
PR = S * (D // 128)

_mesh = plsc.VectorSubcoreMesh(
    core_axis_name="c", subcore_axis_name="s", num_cores=NC, num_subcores=NS
)


def _add_chunk(o_ref, t_ref):
    @plsc.parallel_loop(0, RPC, step=1, unroll=2)
    def body(r):
        for j in range(128 // L):
            sl = pl.ds(j * L, L)
            plsc.addupdate(o_ref.at[r, sl], t_ref[r, sl])


@functools.partial(
    pl.kernel,
    out_type=jax.ShapeDtypeStruct((XR, 128), jnp.float32),
    mesh=_mesh,
    scratch_types=[
        pltpu.VMEM((RPC, 128), jnp.float32),
        pltpu.VMEM((RPC, 128), jnp.float32),
        pltpu.VMEM((RPC, 128), jnp.float32),
        pltpu.SemaphoreType.DMA,
        pltpu.SemaphoreType.DMA,
        pltpu.SemaphoreType.DMA,
        pltpu.SemaphoreType.DMA,
        pltpu.SemaphoreType.DMA,
    ],
)
def _pos_add_sc(x_hbm, pos_hbm, out_hbm, t_ref, o0, o1, st, si0, si1, so0, so1):
    wid = lax.axis_index("s") * NC + lax.axis_index("c")
    base = wid * (ROWS_W * (D // 128))
    PB = PR

    def chunk(c, _):
        off = base + c * RPC
        tin = pltpu.make_async_copy(pos_hbm.at[pl.ds(off, RPC)], t_ref, st)
        in0 = pltpu.make_async_copy(x_hbm.at[pl.ds(0 * PB + off, RPC)], o0, si0)
        in1 = pltpu.make_async_copy(x_hbm.at[pl.ds(1 * PB + off, RPC)], o1, si1)
        in2 = pltpu.make_async_copy(x_hbm.at[pl.ds(2 * PB + off, RPC)], o0, si0)
        in3 = pltpu.make_async_copy(x_hbm.at[pl.ds(3 * PB + off, RPC)], o1, si1)
        out0 = pltpu.make_async_copy(o0, out_hbm.at[pl.ds(0 * PB + off, RPC)], so0)
        out1 = pltpu.make_async_copy(o1, out_hbm.at[pl.ds(1 * PB + off, RPC)], so1)
        out2 = pltpu.make_async_copy(o0, out_hbm.at[pl.ds(2 * PB + off, RPC)], so0)
        out3 = pltpu.make_async_copy(o1, out_hbm.at[pl.ds(3 * PB + off, RPC)], so1)

        tin.start()
        in0.start()
        in1.start()
        tin.wait()
        in0.wait()
        _add_chunk(o0, t_ref)
        out0.start()
        in1.wait()
        _add_chunk(o1, t_ref)
        out1.start()
        out0.wait()
        in2.start()
        out1.wait()
        in3.start()
        in2.wait()
        _add_chunk(o0, t_ref)
        out2.start()
        in3.wait()
        _add_chunk(o1, t_ref)
        out3.start()
        out2.wait()
        out3.wait()
        return 0

    lax.fori_loop(0, N_CHUNKS, chunk, 0, unroll=False)


def kernel(x, pos_table):
    xf = x.reshape(XR, 128)
    pf = pos_table[:S].reshape(PR, 128)
    out = _pos_add_sc(xf, pf)
    return out.reshape(B, S, D)


# speedup vs baseline: 2.2835x; 2.2835x over previous
import functools

import jax
import jax.numpy as jnp
from jax import lax
from jax.experimental import pallas as pl
from jax.experimental.pallas import tpu as pltpu
from jax.experimental.pallas import tpu_sc as plsc

B, S, D = 4, 8192, 1024
NC, NS, L = 2, 16, 16
NW = NC * NS
ROWS_W = S // NW
CH = 16
N_CHUNKS = ROWS_W // CH

_mesh = plsc.VectorSubcoreMesh(
    core_axis_name="c", subcore_axis_name="s", num_cores=NC, num_subcores=NS
)


def _add_chunk(o_ref, t_ref):
    @plsc.parallel_loop(0, CH, step=1, unroll=1)
    def body(r):
        for j in range(D // L):
            sl = pl.ds(j * L, L)
            plsc.addupdate(o_ref.at[r, sl], t_ref[r, sl])


@functools.partial(
    pl.kernel,
    out_type=jax.ShapeDtypeStruct((B, S, D), jnp.float32),
    mesh=_mesh,
    scratch_types=[
        pltpu.VMEM((CH, D), jnp.float32),
        pltpu.VMEM((CH, D), jnp.float32),
        pltpu.VMEM((CH, D), jnp.float32),
        pltpu.SemaphoreType.DMA,
        pltpu.SemaphoreType.DMA,
        pltpu.SemaphoreType.DMA,
        pltpu.SemaphoreType.DMA,
        pltpu.SemaphoreType.DMA,
    ],
    compiler_params=pltpu.CompilerParams(use_tc_tiling_on_sc=True),
)
def _pos_add_sc(x_hbm, pos_hbm, out_hbm, t_ref, o0, o1, st, si0, si1, so0, so1):
    wid = lax.axis_index("s") * NC + lax.axis_index("c")
    base = wid * ROWS_W

    def chunk(c, _):
        row = base + c * CH
        sl = pl.ds(row, CH)
        tin = pltpu.make_async_copy(pos_hbm.at[sl], t_ref, st)
        in0 = pltpu.make_async_copy(x_hbm.at[0, sl], o0, si0)
        in1 = pltpu.make_async_copy(x_hbm.at[1, sl], o1, si1)
        in2 = pltpu.make_async_copy(x_hbm.at[2, sl], o0, si0)
        in3 = pltpu.make_async_copy(x_hbm.at[3, sl], o1, si1)
        out0 = pltpu.make_async_copy(o0, out_hbm.at[0, sl], so0)
        out1 = pltpu.make_async_copy(o1, out_hbm.at[1, sl], so1)
        out2 = pltpu.make_async_copy(o0, out_hbm.at[2, sl], so0)
        out3 = pltpu.make_async_copy(o1, out_hbm.at[3, sl], so1)

        tin.start()
        in0.start()
        in1.start()
        tin.wait()
        in0.wait()
        _add_chunk(o0, t_ref)
        out0.start()
        in1.wait()
        _add_chunk(o1, t_ref)
        out1.start()
        out0.wait()
        in2.start()
        out1.wait()
        in3.start()
        in2.wait()
        _add_chunk(o0, t_ref)
        out2.start()
        in3.wait()
        _add_chunk(o1, t_ref)
        out3.start()
        out2.wait()
        out3.wait()
        return 0

    lax.fori_loop(0, N_CHUNKS, chunk, 0, unroll=False)


def kernel(x, pos_table):
    return _pos_add_sc(x, pos_table[:S])


# speedup vs baseline: 2.8143x; 1.2324x over previous
import functools

import jax
import jax.numpy as jnp
from jax import lax
from jax.experimental import pallas as pl
from jax.experimental.pallas import tpu as pltpu
from jax.experimental.pallas import tpu_sc as plsc

B, S, D = 4, 8192, 1024
NC, NS, L = 2, 16, 16
NW = NC * NS
ROWS_W = S // NW
CH = 16
N_CHUNKS = ROWS_W // CH

_mesh = plsc.VectorSubcoreMesh(
    core_axis_name="c", subcore_axis_name="s", num_cores=NC, num_subcores=NS
)


def _add_chunk(o_ref, t_ref):
    @plsc.parallel_loop(0, CH, step=1, unroll=1)
    def body(r):
        for j in range(D // L):
            sl = pl.ds(j * L, L)
            plsc.addupdate(o_ref.at[r, sl], t_ref[r, sl])


@functools.partial(
    pl.kernel,
    out_type=jax.ShapeDtypeStruct((B, S, D), jnp.float32),
    mesh=_mesh,
    scratch_types=[
        pltpu.VMEM((CH, D), jnp.float32),
        pltpu.VMEM((CH, D), jnp.float32),
        pltpu.VMEM((CH, D), jnp.float32),
        pltpu.VMEM((CH, D), jnp.float32),
        pltpu.VMEM((CH, D), jnp.float32),
        pltpu.VMEM((CH, D), jnp.float32),
        pltpu.SemaphoreType.DMA,
        pltpu.SemaphoreType.DMA,
        pltpu.SemaphoreType.DMA,
        pltpu.SemaphoreType.DMA,
        pltpu.SemaphoreType.DMA,
        pltpu.SemaphoreType.DMA,
        pltpu.SemaphoreType.DMA,
        pltpu.SemaphoreType.DMA,
        pltpu.SemaphoreType.DMA,
        pltpu.SemaphoreType.DMA,
    ],
    compiler_params=pltpu.CompilerParams(use_tc_tiling_on_sc=True),
)
def _pos_add_sc(x_hbm, pos_hbm, out_hbm,
                t0, t1, o0, o1, o2, o3,
                st0, st1, si0, si1, si2, si3, so0, so1, so2, so3):
    wid = lax.axis_index("s") * NC + lax.axis_index("c")
    base = wid * ROWS_W

    def tin(c, t_ref, sem):
        return pltpu.make_async_copy(pos_hbm.at[pl.ds(base + c * CH, CH)], t_ref, sem)

    def xin(b, c, buf, sem):
        return pltpu.make_async_copy(x_hbm.at[b, pl.ds(base + c * CH, CH)], buf, sem)

    def xout(b, c, buf, sem):
        return pltpu.make_async_copy(buf, out_hbm.at[b, pl.ds(base + c * CH, CH)], sem)

    def do_chunk(c, t_ref, sT):
        tin(c, t_ref, sT).wait()
        xin(0, c, o0, si0).wait()
        _add_chunk(o0, t_ref)
        xout(0, c, o0, so0).start()

        @pl.when(c >= 1)
        def _():
            xout(2, c - 1, o2, so2).wait()

        xin(2, c, o2, si2).start()
        xin(1, c, o1, si1).wait()
        _add_chunk(o1, t_ref)
        xout(1, c, o1, so1).start()

        @pl.when(c >= 1)
        def _():
            xout(3, c - 1, o3, so3).wait()

        xin(3, c, o3, si3).start()
        xin(2, c, o2, si2).wait()
        _add_chunk(o2, t_ref)
        xout(2, c, o2, so2).start()

        @pl.when(c + 1 < N_CHUNKS)
        def _():
            xout(0, c, o0, so0).wait()
            xin(0, c + 1, o0, si0).start()

        xin(3, c, o3, si3).wait()
        _add_chunk(o3, t_ref)
        xout(3, c, o3, so3).start()

        @pl.when(c + 1 < N_CHUNKS)
        def _():
            xout(1, c, o1, so1).wait()
            xin(1, c + 1, o1, si1).start()

        @pl.when(c + 2 < N_CHUNKS)
        def _():
            tin(c + 2, t_ref, sT).start()

    tin(0, t0, st0).start()
    tin(1, t1, st1).start()
    xin(0, 0, o0, si0).start()
    xin(1, 0, o1, si1).start()

    def pair(i, _):
        do_chunk(2 * i, t0, st0)
        do_chunk(2 * i + 1, t1, st1)
        return 0

    lax.fori_loop(0, N_CHUNKS // 2, pair, 0, unroll=False)

    xout(0, N_CHUNKS - 1, o0, so0).wait()
    xout(1, N_CHUNKS - 1, o1, so1).wait()
    xout(2, N_CHUNKS - 1, o2, so2).wait()
    xout(3, N_CHUNKS - 1, o3, so3).wait()


def kernel(x, pos_table):
    return _pos_add_sc(x, pos_table[:S])


# speedup vs baseline: 3.6274x; 1.2889x over previous
import functools

import jax
import jax.numpy as jnp
from jax import lax
from jax.experimental import pallas as pl
from jax.experimental.pallas import tpu as pltpu
from jax.experimental.pallas import tpu_sc as plsc

B, S, D = 4, 8192, 1024
NC, NS, L = 2, 16, 16
NW = NC * NS
ROWS_W = S // NW
CH = 8
N_CHUNKS = ROWS_W // CH
N_PAIRS = N_CHUNKS // 2

_mesh = plsc.VectorSubcoreMesh(
    core_axis_name="c", subcore_axis_name="s", num_cores=NC, num_subcores=NS
)


def _add_chunk(o_ref, t_ref):
    @plsc.parallel_loop(0, CH, step=1, unroll=1)
    def body(r):
        for j in range(D // L):
            sl = pl.ds(j * L, L)
            plsc.addupdate(o_ref.at[r, sl], t_ref[r, sl])


@functools.partial(
    pl.kernel,
    out_type=jax.ShapeDtypeStruct((B, S, D), jnp.float32),
    mesh=_mesh,
    scratch_types=[
        [pltpu.VMEM((CH, D), jnp.float32)] * 2,
        [pltpu.VMEM((CH, D), jnp.float32)] * 8,
        [pltpu.SemaphoreType.DMA] * 2,
        [pltpu.SemaphoreType.DMA] * 8,
        [pltpu.SemaphoreType.DMA] * 8,
    ],
    compiler_params=pltpu.CompilerParams(use_tc_tiling_on_sc=True),
)
def _pos_add_sc(x_hbm, pos_hbm, out_hbm, t, o, st, si, so):
    wid = lax.axis_index("s") * NC + lax.axis_index("c")
    base = wid * ROWS_W

    def tin(c, k):
        return pltpu.make_async_copy(pos_hbm.at[pl.ds(base + c * CH, CH)], t[k], st[k])

    def xin(b, c, j):
        return pltpu.make_async_copy(x_hbm.at[b, pl.ds(base + c * CH, CH)], o[j], si[j])

    def xout(b, c, j):
        return pltpu.make_async_copy(o[j], out_hbm.at[b, pl.ds(base + c * CH, CH)], so[j])

    def pair(i, _):
        c0 = 2 * i
        c1 = c0 + 1
        tin(c0, 0).wait()
        for j in range(4):
            xin(j, c0, j).wait()
            _add_chunk(o[j], t[0])
            xout(j, c0, j).start()

            @pl.when(c0 >= 1)
            def _():
                xout(j, c1 - 2, j + 4).wait()

            xin(j, c1, j + 4).start()

        @pl.when(c0 + 2 < N_CHUNKS)
        def _():
            tin(c0 + 2, 0).start()

        tin(c1, 1).wait()
        for j in range(4):
            xin(j, c1, j + 4).wait()
            _add_chunk(o[j + 4], t[1])
            xout(j, c1, j + 4).start()

            @pl.when(c0 + 2 < N_CHUNKS)
            def _():
                xout(j, c0, j).wait()
                xin(j, c0 + 2, j).start()

        @pl.when(c1 + 2 < N_CHUNKS)
        def _():
            tin(c1 + 2, 1).start()

        return 0

    tin(0, 0).start()
    tin(1, 1).start()
    for j in range(4):
        xin(j, 0, j).start()

    lax.fori_loop(0, N_PAIRS, pair, 0, unroll=False)

    for j in range(4):
        xout(j, N_CHUNKS - 2, j).wait()
    for j in range(4):
        xout(j, N_CHUNKS - 1, j + 4).wait()


def kernel(x, pos_table):
    return _pos_add_sc(x, pos_table[:S])
